# baseline (speedup 1.0000x reference)
import functools

import jax
import jax.numpy as jnp
from jax.experimental import pallas as pl
from jax.experimental.pallas import tpu as pltpu


def _pad8(a):
    rows = a.shape[0]
    return jnp.pad(a, ((0, 8 - rows), (0, 0)))


def _prep_body(c0_ref, c1_ref, r_ref, *, W, G):
    iota = jax.lax.broadcasted_iota(jnp.int32, (W, G), 0)
    a0 = c0_ref[0:1, :]
    b0 = c0_ref[1:2, :]
    a1 = c1_ref[0:1, :]
    b1 = c1_ref[1:2, :]
    one = jnp.bfloat16(1)
    two = jnp.bfloat16(2)
    r_ref[:, :G] = ((iota == a0).astype(jnp.bfloat16) * two
                    + (iota == b0).astype(jnp.bfloat16) * one)
    r_ref[:, G:2 * G] = ((iota == a1).astype(jnp.bfloat16) * two
                         + (iota == b1).astype(jnp.bfloat16) * one)
    r_ref[:, 2 * G:] = ((iota == (a1 - W)).astype(jnp.bfloat16) * two
                        + (iota == (b1 - W)).astype(jnp.bfloat16) * one)


def _gator_body(x_ref, r_ref, g0_ref, g1_ref, out_ref, *, W, G):
    xb = x_ref[...]
    xbb = xb.astype(jnp.bfloat16)

    M = jnp.dot(xbb, r_ref[:, :2 * G],
                preferred_element_type=jnp.float32)
    t0 = M[:, :G]

    g0 = g0_ref
    out0 = jnp.where(t0 < 0.5, g0[0:1, :],
            jnp.where(t0 < 1.5, g0[1:2, :],
             jnp.where(t0 < 2.5, g0[2:3, :], g0[3:4, :])))

    t1 = M[:, G:] + jnp.dot(out0.astype(jnp.bfloat16), r_ref[:, 2 * G:],
                            preferred_element_type=jnp.float32)

    g1 = g1_ref
    out1 = jnp.where(t1 < 0.5, g1[0:1, :],
            jnp.where(t1 < 1.5, g1[1:2, :],
             jnp.where(t1 < 2.5, g1[2:3, :], g1[3:4, :])))

    out_ref[:, :W] = xb
    out_ref[:, W:W + G] = out0
    out_ref[:, W + G:] = out1


@jax.jit
def kernel(x, gates0, choices0, gates1, choices1):
    B, W = x.shape
    G = gates0.shape[0]
    TB = 2048

    g0t = _pad8(gates0.T)
    g1t = _pad8(gates1.T)
    c0t = _pad8(choices0.T.astype(jnp.int32))
    c1t = _pad8(choices1.T.astype(jnp.int32))

    rmat = pl.pallas_call(
        functools.partial(_prep_body, W=W, G=G),
        out_shape=jax.ShapeDtypeStruct((W, 3 * G), jnp.bfloat16),
    )(c0t, c1t)

    body = functools.partial(_gator_body, W=W, G=G)
    out = pl.pallas_call(
        body,
        grid=(B // TB,),
        in_specs=[
            pl.BlockSpec((TB, W), lambda i: (i, 0)),
            pl.BlockSpec((W, 3 * G), lambda i: (0, 0)),
            pl.BlockSpec((8, G), lambda i: (0, 0)),
            pl.BlockSpec((8, G), lambda i: (0, 0)),
        ],
        out_specs=pl.BlockSpec((TB, W + 2 * G), lambda i: (i, 0)),
        out_shape=jax.ShapeDtypeStruct((B, W + 2 * G), jnp.float32),
        compiler_params=pltpu.CompilerParams(
            dimension_semantics=("parallel",),
        ),
    )(x, rmat, g0t, g1t)
    return out
